# baseline (speedup 1.0000x reference)
import jax
import jax.numpy as jnp
from jax.experimental import pallas as pl

_B, _S, _H, _NH = 4, 2048, 1024, 16
_HQ = _H // 4
_K = _S // 2


def _dot_t(x, w):
    return jax.lax.dot_general(x, w, (((1,), (1,)), ((), ())),
                               preferred_element_type=jnp.float32)


def _router_body(h_ref, amc_ref, cw0_ref, cw1_ref, cw2_ref,
                 aw1_ref, aw2_ref, ab_ref,
                 r1w_ref, r1b_ref, r2w_ref, r2b_ref,
                 routed_ref, rw_ref):
    n_chunks = 4
    t = _S // n_chunks
    zq = jnp.zeros((1, _HQ), jnp.float32)
    ssum = jnp.zeros((1, _H), jnp.float32)
    sumsq_col_parts = []
    for c in range(n_chunks):
        lo_r = c * t
        hc = h_ref[0, lo_r:lo_r + t, :]

        u1 = jnp.dot(hc, cw1_ref[...], preferred_element_type=jnp.float32)
        if c == 0:
            u0 = jnp.concatenate(
                [zq, jnp.dot(h_ref[0, 0:t - 1, :], cw0_ref[...],
                             preferred_element_type=jnp.float32)], axis=0)
        else:
            u0 = jnp.dot(h_ref[0, lo_r - 1:lo_r + t - 1, :], cw0_ref[...],
                         preferred_element_type=jnp.float32)
        if c == n_chunks - 1:
            u2 = jnp.concatenate(
                [jnp.dot(h_ref[0, lo_r + 1:_S, :], cw2_ref[...],
                         preferred_element_type=jnp.float32), zq], axis=0)
        else:
            u2 = jnp.dot(h_ref[0, lo_r + 1:lo_r + t + 1, :], cw2_ref[...],
                         preferred_element_type=jnp.float32)
        temporal = u0 + u1 + u2

        ce = (_dot_t(hc, aw1_ref[...]) + _dot_t(temporal, aw2_ref[...])
              + ab_ref[...])

        ssum = ssum + jnp.sum(ce, axis=0, keepdims=True)
        sumsq_col_parts.append(jnp.sum(ce * ce, axis=1, keepdims=True))

    ri = ssum * (1.0 / _S)
    hdn = jnp.maximum(_dot_t(ri, r1w_ref[...]) + r1b_ref[...], 0.0)
    logits = _dot_t(hdn, r2w_ref[...]) + r2b_ref[...]
    s1 = jax.nn.softmax(logits, axis=-1)
    rw_ref[0] = jax.nn.softmax(s1, axis=-1)

    sumsq_col = jnp.concatenate(sumsq_col_parts, axis=0)
    amask_col = amc_ref[0] != 0.0
    impsq_col = jnp.where(amask_col, sumsq_col, 0.0)
    bits_col = jax.lax.bitcast_convert_type(impsq_col, jnp.int32)
    idx_col = jax.lax.broadcasted_iota(jnp.int32, (_S, 1), 0)
    jlane = jax.lax.broadcasted_iota(jnp.int32, (1, 128), 1)

    thresh = jnp.int32(0)
    for shift in (24, 17, 10, 3, 0):
        cand = thresh + (jlane << shift)
        cnt = jnp.sum((bits_col >= cand).astype(jnp.int32),
                      axis=0, keepdims=True)
        j = jnp.max(jnp.where(cnt >= _K, jlane, 0))
        thresh = thresh + (j << shift)

    gt_col = bits_col > thresh
    eq_col = bits_col == thresh
    need = _K - jnp.sum(gt_col.astype(jnp.int32))

    cnt1 = jnp.sum((eq_col & (idx_col <= jlane * 16 + 15)).astype(jnp.int32),
                   axis=0, keepdims=True)
    j0 = jnp.min(jnp.where(cnt1 >= need, jlane, 127))
    cnt2 = jnp.sum((eq_col & (idx_col <= j0 * 16 + jlane)).astype(jnp.int32),
                   axis=0, keepdims=True)
    t_idx = j0 * 16 + jnp.min(jnp.where(cnt2 >= need, jlane, 127))

    mask_col = (gt_col | (eq_col & (idx_col <= t_idx))) & amask_col
    maskf = mask_col.astype(jnp.float32)
    for c in range(n_chunks):
        lo_r = c * t
        routed_ref[0, lo_r:lo_r + t, :] = (
            h_ref[0, lo_r:lo_r + t, :] * maskf[lo_r:lo_r + t])


def kernel(hidden_states, attention_mask, conv_w, conv_b, agg_w, agg_b,
           r1_w, r1_b, r2_w, r2_b):
    f32 = jnp.float32
    ab2 = (agg_b + jnp.dot(agg_w[:, _H:], conv_b))[None, :]
    amc = attention_mask[:, :, None].astype(f32)

    routed, rw = pl.pallas_call(
        _router_body,
        grid=(_B,),
        in_specs=[
            pl.BlockSpec((1, _S, _H), lambda b: (b, 0, 0)),
            pl.BlockSpec((1, _S, 1), lambda b: (b, 0, 0)),
            pl.BlockSpec((_H, _HQ), lambda b: (0, 0)),
            pl.BlockSpec((_H, _HQ), lambda b: (0, 0)),
            pl.BlockSpec((_H, _HQ), lambda b: (0, 0)),
            pl.BlockSpec((_H, _H), lambda b: (0, 0)),
            pl.BlockSpec((_H, _HQ), lambda b: (0, 4)),
            pl.BlockSpec((1, _H), lambda b: (0, 0)),
            pl.BlockSpec((_H // 2, _H), lambda b: (0, 0)),
            pl.BlockSpec((1, _H // 2), lambda b: (0, 0)),
            pl.BlockSpec((_NH, _H // 2), lambda b: (0, 0)),
            pl.BlockSpec((1, _NH), lambda b: (0, 0)),
        ],
        out_specs=[
            pl.BlockSpec((1, _S, _H), lambda b: (b, 0, 0)),
            pl.BlockSpec((1, 1, _NH), lambda b: (b, 0, 0)),
        ],
        out_shape=[
            jax.ShapeDtypeStruct((_B, _S, _H), f32),
            jax.ShapeDtypeStruct((_B, 1, _NH), f32),
        ],
    )(hidden_states.astype(f32), amc,
      conv_w[:, :, 0].T, conv_w[:, :, 1].T, conv_w[:, :, 2].T,
      agg_w, agg_w, ab2, r1_w, r1_b[None, :], r2_w, r2_b[None, :])

    return routed, rw[:, 0, :]


# speedup vs baseline: 1.0099x; 1.0099x over previous
import jax
import jax.numpy as jnp
from jax.experimental import pallas as pl

_B, _S, _H, _NH = 4, 2048, 1024, 16
_HQ = _H // 4
_K = _S // 2


def _dot_t(x, w):
    return jax.lax.dot_general(x, w, (((1,), (1,)), ((), ())),
                               preferred_element_type=jnp.float32)


def _router_body(h_ref, amc_ref, cw_ref, aw1_ref, aw2_ref, ab_ref,
                 r1w_ref, r1b_ref, r2w_ref, r2b_ref,
                 routed_ref, rw_ref):
    n_chunks = 4
    t = _S // n_chunks
    zrow = jnp.zeros((1, _H), jnp.float32)
    ssum = jnp.zeros((1, _H), jnp.float32)
    sumsq_col_parts = []
    for c in range(n_chunks):
        lo_r = c * t
        if c == 0:
            hs = jnp.concatenate([zrow, h_ref[0, 0:t + 1, :]], axis=0)
        elif c == n_chunks - 1:
            hs = jnp.concatenate([h_ref[0, lo_r - 1:_S, :], zrow], axis=0)
        else:
            hs = h_ref[0, lo_r - 1:lo_r + t + 1, :]

        u = jnp.dot(hs, cw_ref[...], preferred_element_type=jnp.float32)
        temporal = (u[0:t, 0:_HQ] + u[1:t + 1, _HQ:2 * _HQ]
                    + u[2:t + 2, 2 * _HQ:3 * _HQ])

        ce = (_dot_t(hs[1:t + 1], aw1_ref[...])
              + _dot_t(temporal, aw2_ref[...])
              + ab_ref[...])

        ssum = ssum + jnp.sum(ce, axis=0, keepdims=True)
        sumsq_col_parts.append(jnp.sum(ce * ce, axis=1, keepdims=True))

    ri = ssum * (1.0 / _S)
    hdn = jnp.maximum(_dot_t(ri, r1w_ref[...]) + r1b_ref[...], 0.0)
    logits = _dot_t(hdn, r2w_ref[...]) + r2b_ref[...]
    s1 = jax.nn.softmax(logits, axis=-1)
    rw_ref[0] = jax.nn.softmax(s1, axis=-1)

    sumsq_col = jnp.concatenate(sumsq_col_parts, axis=0)
    amask_col = amc_ref[0] != 0.0
    impsq_col = jnp.where(amask_col, sumsq_col, 0.0)
    bits_col = jax.lax.bitcast_convert_type(impsq_col, jnp.int32)
    idx_col = jax.lax.broadcasted_iota(jnp.int32, (_S, 1), 0)
    jlane = jax.lax.broadcasted_iota(jnp.int32, (1, 128), 1)

    thresh = jnp.int32(0)
    for shift in (24, 17, 10, 3, 0):
        cand = thresh + (jlane << shift)
        cnt = jnp.sum((bits_col >= cand).astype(jnp.int32),
                      axis=0, keepdims=True)
        j = jnp.max(jnp.where(cnt >= _K, jlane, 0))
        thresh = thresh + (j << shift)

    gt_col = bits_col > thresh
    eq_col = bits_col == thresh
    need = _K - jnp.sum(gt_col.astype(jnp.int32))

    cnt1 = jnp.sum((eq_col & (idx_col <= jlane * 16 + 15)).astype(jnp.int32),
                   axis=0, keepdims=True)
    j0 = jnp.min(jnp.where(cnt1 >= need, jlane, 127))
    cnt2 = jnp.sum((eq_col & (idx_col <= j0 * 16 + jlane)).astype(jnp.int32),
                   axis=0, keepdims=True)
    t_idx = j0 * 16 + jnp.min(jnp.where(cnt2 >= need, jlane, 127))

    mask_col = (gt_col | (eq_col & (idx_col <= t_idx))) & amask_col
    maskf = mask_col.astype(jnp.float32)
    for c in range(n_chunks):
        lo_r = c * t
        routed_ref[0, lo_r:lo_r + t, :] = (
            h_ref[0, lo_r:lo_r + t, :] * maskf[lo_r:lo_r + t])


def kernel(hidden_states, attention_mask, conv_w, conv_b, agg_w, agg_b,
           r1_w, r1_b, r2_w, r2_b):
    f32 = jnp.float32
    ab2 = (agg_b + jnp.dot(agg_w[:, _H:], conv_b))[None, :]
    amc = attention_mask[:, :, None].astype(f32)

    routed, rw = pl.pallas_call(
        _router_body,
        grid=(_B,),
        in_specs=[
            pl.BlockSpec((1, _S, _H), lambda b: (b, 0, 0)),
            pl.BlockSpec((1, _S, 1), lambda b: (b, 0, 0)),
            pl.BlockSpec((_H, 3 * _HQ), lambda b: (0, 0)),
            pl.BlockSpec((_H, _H), lambda b: (0, 0)),
            pl.BlockSpec((_H, _HQ), lambda b: (0, 4)),
            pl.BlockSpec((1, _H), lambda b: (0, 0)),
            pl.BlockSpec((_H // 2, _H), lambda b: (0, 0)),
            pl.BlockSpec((1, _H // 2), lambda b: (0, 0)),
            pl.BlockSpec((_NH, _H // 2), lambda b: (0, 0)),
            pl.BlockSpec((1, _NH), lambda b: (0, 0)),
        ],
        out_specs=[
            pl.BlockSpec((1, _S, _H), lambda b: (b, 0, 0)),
            pl.BlockSpec((1, 1, _NH), lambda b: (b, 0, 0)),
        ],
        out_shape=[
            jax.ShapeDtypeStruct((_B, _S, _H), f32),
            jax.ShapeDtypeStruct((_B, 1, _NH), f32),
        ],
    )(hidden_states.astype(f32), amc,
      jnp.concatenate([conv_w[:, :, 0].T, conv_w[:, :, 1].T,
                       conv_w[:, :, 2].T], axis=1),
      agg_w, agg_w, ab2, r1_w, r1_b[None, :], r2_w, r2_b[None, :])

    return routed, rw[:, 0, :]
